# baseline (speedup 1.0000x reference)
import functools
import math

import jax
import jax.numpy as jnp
from jax.experimental import pallas as pl
import jax.experimental.pallas.tpu as pltpu

D_TIME = 2048
NUM_EXPERTS = 4
N_FOURIER = 32
N_GAUSS = 32
N_WAVELET = 32
SPLINE_NUM = 5
SPLINE_K = 3
BN = 256
K_FEAT = 144
K_AUG = 152
H_W = 256

_NEG_INF = float("-inf")


def _moe_block(t_ref, wr_ref, br_ref, a_ref, b0_ref, m_ref, h_ref,
               wbig_ref, out_ref, raw_ref, mask_ref):
    x = t_ref[:, :]

    logits = x * wr_ref[:, :] + br_ref[:, :]
    mx = jnp.max(logits, axis=-1, keepdims=True)
    ex = jnp.exp(logits - mx)
    raw = ex / jnp.sum(ex, axis=-1, keepdims=True)
    raw_ref[:, :] = raw

    col4 = jax.lax.broadcasted_iota(jnp.int32, raw.shape, 1)
    v1 = jnp.max(raw, axis=-1, keepdims=True)
    i1 = jnp.min(jnp.where(raw == v1, col4, NUM_EXPERTS), axis=-1, keepdims=True)
    raw2 = jnp.where(col4 == i1, _NEG_INF, raw)
    v2 = jnp.max(raw2, axis=-1, keepdims=True)
    i2 = jnp.min(jnp.where(raw2 == v2, col4, NUM_EXPERTS), axis=-1, keepdims=True)

    is1 = col4 == i1
    is2 = col4 == i2
    mask_ref[:, :] = (is1 | is2).astype(jnp.int8)

    e2 = jnp.exp(v2 - v1)
    w1 = 1.0 / (1.0 + e2)
    w2 = e2 / (1.0 + e2)
    disp = jnp.where(is1, w1, 0.0) + jnp.where(is2, w2, 0.0)

    z = x * a_ref[:, :] + b0_ref[:, :]
    s = jnp.sin(z)
    z2 = z * z
    e = jnp.exp(z2 * m_ref[:, :])
    wavelet = (z2 - 1.0) * e
    lane = jax.lax.broadcasted_iota(jnp.int32, z.shape, 1)
    feat1 = jnp.where(lane < 64, s, jnp.where(lane < 96, e, wavelet))
    wsel = jnp.where(lane < 64, disp[:, 0:1],
                     jnp.where(lane < 96, disp[:, 2:3], disp[:, 3:4]))
    feat1 = feat1 * wsel

    h = 2.0 / SPLINE_NUM
    n_knots = SPLINE_NUM + 2 * SPLINE_K + 1

    def knots(offset, length):
        i = jax.lax.broadcasted_iota(
            jnp.int32, (1, length), 1).astype(jnp.float32)
        return (i + (offset - SPLINE_K)) * h - 1.0

    bases = ((x >= knots(0, n_knots - 1)) &
             (x < knots(1, n_knots - 1))).astype(jnp.float32)
    for d in range(1, SPLINE_K + 1):
        width = n_knots - 1 - d
        left = (x - knots(0, width)) * (1.0 / (d * h)) * bases[:, :-1]
        right = (knots(d + 1, width) - x) * (1.0 / (d * h)) * bases[:, 1:]
        bases = left + right
    silu = x * jax.nn.sigmoid(x)
    w_s = disp[:, 1:2]
    feat2 = jnp.concatenate(
        [silu * w_s, bases * w_s,
         jnp.zeros((x.shape[0], K_FEAT - 128 - 9), jnp.float32)], axis=-1)

    feat = jnp.concatenate([feat1, feat2], axis=-1)

    y2 = jnp.dot(feat, h_ref[:, :], preferred_element_type=jnp.float32)
    mu = y2[:, K_FEAT:K_FEAT + 1] * (1.0 / D_TIME)
    sumsq = jnp.sum(feat * y2[:, :K_FEAT], axis=-1, keepdims=True)
    var = sumsq * (1.0 / D_TIME) - mu * mu
    s = jax.lax.rsqrt(var + 1e-5)

    feat_aug = jnp.concatenate(
        [feat * s, -s * mu, jnp.ones((x.shape[0], 1), jnp.float32),
         jnp.zeros((x.shape[0], K_AUG - K_FEAT - 2), jnp.float32)], axis=-1)
    out_ref[:, :] = jnp.dot(feat_aug, wbig_ref[:, :],
                            preferred_element_type=jnp.float32)


@functools.partial(jax.jit, static_argnames=())
def kernel(timestamp_input, W_router, b_router, W_fourier, gauss_centers,
           W_gauss, wave_trans, wave_scales, W_wave, spline_coef, base_w,
           ln_gamma, ln_beta):
    n = timestamp_input.shape[0]
    f32 = jnp.float32

    freqs = jnp.arange(1, N_FOURIER + 1, dtype=f32)
    inv_scale = 1.0 / wave_scales
    lane_a = jnp.concatenate(
        [freqs, freqs, jnp.ones((N_GAUSS,), f32), inv_scale])[None, :]
    lane_b = jnp.concatenate(
        [jnp.zeros((N_FOURIER,), f32),
         jnp.full((N_FOURIER,), math.pi / 2, f32),
         -gauss_centers,
         -wave_trans * inv_scale])[None, :]
    lane_m = jnp.concatenate(
        [jnp.zeros((2 * N_FOURIER,), f32),
         jnp.full((N_GAUSS,), -1.0, f32),
         jnp.full((N_WAVELET,), -0.5, f32)])[None, :]

    w_all = jnp.concatenate(
        [W_fourier[:N_FOURIER], W_fourier[N_FOURIER:], W_gauss, W_wave,
         base_w, spline_coef,
         jnp.zeros((K_FEAT - 128 - 9, D_TIME), f32)], axis=0)

    gram = jnp.dot(w_all, w_all.T,
                   precision=jax.lax.Precision.HIGHEST)
    w_rowsum = jnp.sum(w_all, axis=1, keepdims=True)
    h_mat = jnp.concatenate(
        [gram, w_rowsum, jnp.zeros((K_FEAT, H_W - K_FEAT - 1), f32)], axis=1)
    w_big = jnp.concatenate(
        [w_all * ln_gamma[None, :], ln_gamma[None, :], ln_beta[None, :],
         jnp.zeros((K_AUG - K_FEAT - 2, D_TIME), f32)], axis=0)

    grid = (n // BN,)
    full = lambda shape: pl.BlockSpec(shape, lambda i: (0, 0))
    row = lambda w: pl.BlockSpec((BN, w), lambda i: (i, 0))

    out, raw, mask8 = pl.pallas_call(
        _moe_block,
        grid=grid,
        in_specs=[
            row(1),
            full((1, NUM_EXPERTS)),
            full((1, NUM_EXPERTS)),
            full((1, 128)),
            full((1, 128)),
            full((1, 128)),
            full((K_FEAT, H_W)),
            full((K_AUG, D_TIME)),
        ],
        out_specs=[row(D_TIME), row(NUM_EXPERTS), row(NUM_EXPERTS)],
        compiler_params=pltpu.CompilerParams(
            dimension_semantics=("parallel",)),
        out_shape=[
            jax.ShapeDtypeStruct((n, D_TIME), f32),
            jax.ShapeDtypeStruct((n, NUM_EXPERTS), f32),
            jax.ShapeDtypeStruct((n, NUM_EXPERTS), jnp.int8),
        ],
    )(
        timestamp_input,
        W_router,
        b_router.reshape(1, NUM_EXPERTS),
        lane_a,
        lane_b,
        lane_m,
        h_mat,
        w_big,
    )
    return out, raw, mask8.astype(jnp.bool_)


# speedup vs baseline: 1.3197x; 1.3197x over previous
import functools
import math

import jax
import jax.numpy as jnp
from jax.experimental import pallas as pl
import jax.experimental.pallas.tpu as pltpu
from jax.experimental.pallas import tpu_sc as plsc

D_TIME = 2048
NUM_EXPERTS = 4
N_FOURIER = 32
N_GAUSS = 32
N_WAVELET = 32
SPLINE_NUM = 5
SPLINE_K = 3
BN = 512
K_FEAT = 144
K_AUG = 152
H_W = 256

_NEG_INF = float("-inf")

_SC_NC = 2
_SC_NS = 16
_SC_NW = _SC_NC * _SC_NS
_SC_L = 16


def _router_sc(x_hbm, p_hbm, raw_hbm, mask_hbm, xv, pv, rawv, maskv):
    wid = jax.lax.axis_index("s") * _SC_NC + jax.lax.axis_index("c")
    chunk = xv.shape[0]
    pltpu.sync_copy(x_hbm.at[pl.ds(wid * chunk, chunk)], xv)
    pltpu.sync_copy(p_hbm, pv)
    w = [pv[e, :] for e in range(NUM_EXPERTS)]
    b = [pv[NUM_EXPERTS + e, :] for e in range(NUM_EXPERTS)]

    def body(i, carry):
        sl = pl.ds(i * _SC_L, _SC_L)
        x = xv[sl]
        l = [x * w[e] + b[e] for e in range(NUM_EXPERTS)]
        m = jnp.maximum(jnp.maximum(l[0], l[1]), jnp.maximum(l[2], l[3]))
        ex = [jnp.exp(l[e] - m) for e in range(NUM_EXPERTS)]
        tot = (ex[0] + ex[1]) + (ex[2] + ex[3])
        r = [ex[e] / tot for e in range(NUM_EXPERTS)]
        v1 = jnp.maximum(jnp.maximum(r[0], r[1]), jnp.maximum(r[2], r[3]))
        i1 = jnp.where(r[0] == v1, 0.0,
                       jnp.where(r[1] == v1, 1.0,
                                 jnp.where(r[2] == v1, 2.0, 3.0)))
        q = [jnp.where(i1 == float(e), -1.0, r[e])
             for e in range(NUM_EXPERTS)]
        v2 = jnp.maximum(jnp.maximum(q[0], q[1]), jnp.maximum(q[2], q[3]))
        i2 = jnp.where(q[0] == v2, 0.0,
                       jnp.where(q[1] == v2, 1.0,
                                 jnp.where(q[2] == v2, 2.0, 3.0)))
        for e in range(NUM_EXPERTS):
            rawv[e, sl] = r[e]
            maskv[e, sl] = jnp.where(
                (i1 == float(e)) | (i2 == float(e)), 1.0, 0.0)
        return carry

    jax.lax.fori_loop(0, chunk // _SC_L, body, 0)
    pltpu.sync_copy(rawv, raw_hbm.at[wid])
    pltpu.sync_copy(maskv, mask_hbm.at[wid])


def _moe_block(t_ref, wr_ref, br_ref, a_ref, b0_ref, m_ref, h_ref,
               wbig_ref, out_ref):
    x = t_ref[:, :]

    logits = x * wr_ref[:, :] + br_ref[:, :]
    mx = jnp.max(logits, axis=-1, keepdims=True)
    ex = jnp.exp(logits - mx)
    raw = ex / jnp.sum(ex, axis=-1, keepdims=True)

    col4 = jax.lax.broadcasted_iota(jnp.int32, raw.shape, 1)
    v1 = jnp.max(raw, axis=-1, keepdims=True)
    i1 = jnp.min(jnp.where(raw == v1, col4, NUM_EXPERTS), axis=-1, keepdims=True)
    raw2 = jnp.where(col4 == i1, _NEG_INF, raw)
    v2 = jnp.max(raw2, axis=-1, keepdims=True)
    i2 = jnp.min(jnp.where(raw2 == v2, col4, NUM_EXPERTS), axis=-1, keepdims=True)

    is1 = col4 == i1
    is2 = col4 == i2

    e2 = jnp.exp(v2 - v1)
    w1 = 1.0 / (1.0 + e2)
    w2 = e2 / (1.0 + e2)
    disp = jnp.where(is1, w1, 0.0) + jnp.where(is2, w2, 0.0)

    z = x * a_ref[:, :] + b0_ref[:, :]
    s = jnp.sin(z)
    z2 = z * z
    e = jnp.exp(z2 * m_ref[:, :])
    wavelet = (z2 - 1.0) * e
    lane = jax.lax.broadcasted_iota(jnp.int32, z.shape, 1)
    feat1 = jnp.where(lane < 64, s, jnp.where(lane < 96, e, wavelet))
    wsel = jnp.where(lane < 64, disp[:, 0:1],
                     jnp.where(lane < 96, disp[:, 2:3], disp[:, 3:4]))
    feat1 = feat1 * wsel

    h = 2.0 / SPLINE_NUM
    n_knots = SPLINE_NUM + 2 * SPLINE_K + 1

    def knots(offset, length):
        i = jax.lax.broadcasted_iota(
            jnp.int32, (1, length), 1).astype(jnp.float32)
        return (i + (offset - SPLINE_K)) * h - 1.0

    bases = ((x >= knots(0, n_knots - 1)) &
             (x < knots(1, n_knots - 1))).astype(jnp.float32)
    for d in range(1, SPLINE_K + 1):
        width = n_knots - 1 - d
        left = (x - knots(0, width)) * (1.0 / (d * h)) * bases[:, :-1]
        right = (knots(d + 1, width) - x) * (1.0 / (d * h)) * bases[:, 1:]
        bases = left + right
    silu = x * jax.nn.sigmoid(x)
    w_s = disp[:, 1:2]
    feat2 = jnp.concatenate(
        [silu * w_s, bases * w_s,
         jnp.zeros((x.shape[0], K_FEAT - 128 - 9), jnp.float32)], axis=-1)

    feat = jnp.concatenate([feat1, feat2], axis=-1)

    y2 = jnp.dot(feat, h_ref[:, :], preferred_element_type=jnp.float32)
    mu = y2[:, K_FEAT:K_FEAT + 1] * (1.0 / D_TIME)
    sumsq = jnp.sum(feat * y2[:, :K_FEAT], axis=-1, keepdims=True)
    var = sumsq * (1.0 / D_TIME) - mu * mu
    s = jax.lax.rsqrt(var + 1e-5)

    feat_aug = jnp.concatenate(
        [feat * s, -s * mu, jnp.ones((x.shape[0], 1), jnp.float32),
         jnp.zeros((x.shape[0], K_AUG - K_FEAT - 2), jnp.float32)], axis=-1)
    out_ref[:, :] = jnp.dot(feat_aug, wbig_ref[:, :],
                            preferred_element_type=jnp.float32)


@functools.partial(jax.jit, static_argnames=())
def kernel(timestamp_input, W_router, b_router, W_fourier, gauss_centers,
           W_gauss, wave_trans, wave_scales, W_wave, spline_coef, base_w,
           ln_gamma, ln_beta):
    n = timestamp_input.shape[0]
    f32 = jnp.float32

    freqs = jnp.arange(1, N_FOURIER + 1, dtype=f32)
    inv_scale = 1.0 / wave_scales
    lane_a = jnp.concatenate(
        [freqs, freqs, jnp.ones((N_GAUSS,), f32), inv_scale])[None, :]
    lane_b = jnp.concatenate(
        [jnp.zeros((N_FOURIER,), f32),
         jnp.full((N_FOURIER,), math.pi / 2, f32),
         -gauss_centers,
         -wave_trans * inv_scale])[None, :]
    lane_m = jnp.concatenate(
        [jnp.zeros((2 * N_FOURIER,), f32),
         jnp.full((N_GAUSS,), -1.0, f32),
         jnp.full((N_WAVELET,), -0.5, f32)])[None, :]

    w_all = jnp.concatenate(
        [W_fourier[:N_FOURIER], W_fourier[N_FOURIER:], W_gauss, W_wave,
         base_w, spline_coef,
         jnp.zeros((K_FEAT - 128 - 9, D_TIME), f32)], axis=0)

    gram = jnp.dot(w_all, w_all.T,
                   precision=jax.lax.Precision.HIGHEST)
    w_rowsum = jnp.sum(w_all, axis=1, keepdims=True)
    h_mat = jnp.concatenate(
        [gram, w_rowsum, jnp.zeros((K_FEAT, H_W - K_FEAT - 1), f32)], axis=1)
    w_big = jnp.concatenate(
        [w_all * ln_gamma[None, :], ln_gamma[None, :], ln_beta[None, :],
         jnp.zeros((K_AUG - K_FEAT - 2, D_TIME), f32)], axis=0)

    grid = (n // BN,)
    full = lambda shape: pl.BlockSpec(shape, lambda i: (0, 0))
    row = lambda w: pl.BlockSpec((BN, w), lambda i: (i, 0))

    out = pl.pallas_call(
        _moe_block,
        grid=grid,
        in_specs=[
            row(1),
            full((1, NUM_EXPERTS)),
            full((1, NUM_EXPERTS)),
            full((1, 128)),
            full((1, 128)),
            full((1, 128)),
            full((K_FEAT, H_W)),
            full((K_AUG, D_TIME)),
        ],
        out_specs=row(D_TIME),
        compiler_params=pltpu.CompilerParams(
            dimension_semantics=("parallel",)),
        out_shape=jax.ShapeDtypeStruct((n, D_TIME), f32),
    )(
        timestamp_input,
        W_router,
        b_router.reshape(1, NUM_EXPERTS),
        lane_a,
        lane_b,
        lane_m,
        h_mat,
        w_big,
    )

    chunk = n // _SC_NW
    p_router = jnp.concatenate(
        [jnp.repeat(W_router.reshape(NUM_EXPERTS, 1), _SC_L, axis=1),
         jnp.repeat(b_router.reshape(NUM_EXPERTS, 1), _SC_L, axis=1)], axis=0)
    router = functools.partial(
        pl.kernel,
        mesh=plsc.VectorSubcoreMesh(core_axis_name="c", subcore_axis_name="s"),
        out_type=[
            jax.ShapeDtypeStruct((_SC_NW, NUM_EXPERTS, chunk), f32),
            jax.ShapeDtypeStruct((_SC_NW, NUM_EXPERTS, chunk), f32),
        ],
        scratch_types=[
            pltpu.VMEM((chunk,), f32),
            pltpu.VMEM((2 * NUM_EXPERTS, _SC_L), f32),
            pltpu.VMEM((NUM_EXPERTS, chunk), f32),
            pltpu.VMEM((NUM_EXPERTS, chunk), f32),
        ],
    )(_router_sc)
    raw3, mask3 = router(timestamp_input.reshape(n), p_router)
    raw = jnp.transpose(raw3, (0, 2, 1)).reshape(n, NUM_EXPERTS)
    mask = jnp.transpose(mask3, (0, 2, 1)).reshape(n, NUM_EXPERTS) != 0.0
    return out, raw, mask
